# baseline (speedup 1.0000x reference)
import functools

import jax
import jax.numpy as jnp
from jax import lax
from jax.experimental import pallas as pl
from jax.experimental.pallas import tpu as pltpu
from jax.experimental.pallas import tpu_sc as plsc

VOCAB = 1000000
EMBED_DIM = 32
SEQ = 200
BATCH = 4096

_INFO = plsc.get_sparse_core_info()
_NC, _NS, _L = _INFO.num_cores, _INFO.num_subcores, _INFO.num_lanes
_NW = _NC * _NS
_BBLK = BATCH // _NW
_NG = 4

_NCOL = VOCAB // 128
_CPW = _NCOL // _NW + 1


def _repack_kernel(tT_hbm, tail4_hbm, t4_hbm, src_v, dst_v, tail_v,
                   isem, osem0, osem1):
    wid = lax.axis_index("s") * _NC + lax.axis_index("c")
    osems = (osem0, osem1)
    tok16 = lax.iota(jnp.int32, _L)

    def shuffle(s, width):
        for qq in range(width // 4):
            for j in range(4):
                cols = jnp.full((_L,), 4 * qq + j, jnp.int32)
                for h in range(2):
                    dst_v[s, qq, pl.ds(32 * j + h * _L, _L)] = (
                        plsc.load_gather(src_v.at[s],
                                         [tok16 + h * _L, cols]))

    def wait_out(s):
        pltpu.make_async_copy(
            dst_v.at[s], t4_hbm.at[pl.ds(0, 32)], osems[s]).wait()

    def body(k, _):
        c = pl.multiple_of(k * _NW + wid, 1)
        s = lax.rem(k, 2)

        @pl.when(c < _NCOL)
        def _():
            for sb in range(2):
                @pl.when(s == sb)
                def _():
                    pltpu.async_copy(
                        tT_hbm.at[:, pl.ds(c * 128, 128)],
                        src_v.at[sb], isem).wait()
                    @pl.when(k >= 2)
                    def _():
                        wait_out(sb)
                    shuffle(sb, 128)
                    pltpu.async_copy(
                        dst_v.at[sb], t4_hbm.at[pl.ds(c * 32, 32)], osems[sb])
        return 0

    lax.fori_loop(0, _CPW, body, 0)
    @pl.when(wid == 0)
    def _():
        pltpu.sync_copy(tail4_hbm, tail_v)
        wait_out(0)
        pltpu.sync_copy(tail_v, t4_hbm.at[pl.ds(_NCOL * 32, 16)])
    @pl.when(wid != 0)
    def _():
        wait_out(0)
    wait_out(1)


def _emb_kernel(qT_hbm, cbT_hbm, table4_hbm, out_hbm,
                q_v, cb_v, g_v, o_v,
                gsem0, gsem1, gsem2, gsem3, osem0, osem1):
    wid = lax.axis_index("s") * _NC + lax.axis_index("c")
    col0 = wid * _BBLK
    gsems = (gsem0, gsem1, gsem2, gsem3)
    osems = (osem0, osem1)
    tok16 = lax.iota(jnp.int32, _L)

    pltpu.sync_copy(qT_hbm.at[:, pl.ds(col0, _BBLK)], q_v)
    pltpu.sync_copy(cbT_hbm.at[:, pl.ds(col0, _BBLK)], cb_v)

    def fire_gather(t, s):
        pltpu.async_copy(table4_hbm.at[q_v.at[t]], g_v.at[s], gsems[s])

    def wait_gather(s):
        pltpu.make_async_copy(
            table4_hbm.at[q_v.at[0]], g_v.at[s], gsems[s]).wait()

    def wait_out(s):
        pltpu.make_async_copy(
            o_v.at[s], out_hbm.at[0, :, pl.ds(col0, _BBLK)], osems[s]
        ).wait()

    for t in range(_NG - 1):
        fire_gather(t, t)

    def body(i, _):
        for b in range(_NG):
            t = i * _NG + b
            wait_gather(b)
            @pl.when(t + _NG - 1 < SEQ)
            def _():
                fire_gather(t + _NG - 1, (b + _NG - 1) % _NG)
            so = b % 2
            @pl.when(t >= 2)
            def _():
                wait_out(so)
            for tb in range(_BBLK // _L):
                cb = cb_v[t, pl.ds(tb * _L, _L)]
                rows = tok16 + (tb * _L)
                for d in range(EMBED_DIM):
                    o_v[so, d, pl.ds(tb * _L, _L)] = plsc.load_gather(
                        g_v.at[b], [rows, cb + d])
            pltpu.async_copy(
                o_v.at[so], out_hbm.at[t, :, pl.ds(col0, _BBLK)], osems[so]
            )
        return 0

    lax.fori_loop(0, SEQ // _NG, body, 0)
    wait_out(0)
    wait_out(1)


def kernel(x, table):
    assert x.shape == (BATCH, SEQ) and table.shape == (VOCAB, EMBED_DIM)
    xi = x.astype(jnp.int32)
    qT = (xi >> 2).T
    cbT = ((xi & 3) * EMBED_DIM).T
    tT = table.T

    mesh = plsc.VectorSubcoreMesh(core_axis_name="c", subcore_axis_name="s")
    cp = pltpu.CompilerParams(use_tc_tiling_on_sc=True,
                              needs_layout_passes=False)

    ka = functools.partial(
        pl.kernel, mesh=mesh,
        out_type=jax.ShapeDtypeStruct((VOCAB // 4, 128), jnp.float32),
        scratch_types=[
            pltpu.VMEM((2, EMBED_DIM, 128), jnp.float32),
            pltpu.VMEM((2, 32, 128), jnp.float32),
            pltpu.VMEM((16, 128), jnp.float32),
            pltpu.SemaphoreType.DMA,
            pltpu.SemaphoreType.DMA,
            pltpu.SemaphoreType.DMA,
        ],
        compiler_params=cp,
    )(_repack_kernel)

    kb = functools.partial(
        pl.kernel, mesh=mesh,
        out_type=jax.ShapeDtypeStruct((SEQ, EMBED_DIM, BATCH), jnp.float32),
        scratch_types=[
            pltpu.VMEM((SEQ, _BBLK), jnp.int32),
            pltpu.VMEM((SEQ, _BBLK), jnp.int32),
            pltpu.VMEM((_NG, _BBLK, 128), jnp.float32),
            pltpu.VMEM((2, EMBED_DIM, _BBLK), jnp.float32),
            pltpu.SemaphoreType.DMA,
            pltpu.SemaphoreType.DMA,
            pltpu.SemaphoreType.DMA,
            pltpu.SemaphoreType.DMA,
            pltpu.SemaphoreType.DMA,
            pltpu.SemaphoreType.DMA,
        ],
        compiler_params=cp,
    )(_emb_kernel)

    tail4 = table[_NCOL * 128:].reshape(16, 128)
    table4 = ka(tT, tail4)
    out = kb(qT, cbT, table4)
    return out.transpose(2, 0, 1)


# speedup vs baseline: 1.1350x; 1.1350x over previous
import functools

import jax
import jax.numpy as jnp
from jax import lax
from jax.experimental import pallas as pl
from jax.experimental.pallas import tpu as pltpu
from jax.experimental.pallas import tpu_sc as plsc

VOCAB = 1000000
EMBED_DIM = 32
SEQ = 200
BATCH = 4096

_INFO = plsc.get_sparse_core_info()
_NC, _NS, _L = _INFO.num_cores, _INFO.num_subcores, _INFO.num_lanes
_NW = _NC * _NS
_BBLK = BATCH // _NW
_NG = 4

_NSC = VOCAB // 512
_SCPW = _NSC // _NW + 1


def _repack_kernel(tT_hbm, tail4_hbm, t4_hbm, src_v, dst_v, tail_v,
                   isem0, isem1, osem0, osem1):
    wid = lax.axis_index("s") * _NC + lax.axis_index("c")
    isems = (isem0, isem1)
    osems = (osem0, osem1)
    tok16 = lax.iota(jnp.int32, _L)

    def fire_in(c, s):
        pltpu.async_copy(
            tT_hbm.at[:, pl.ds(c * 512, 512)], src_v.at[s], isems[s])

    def wait_in(s):
        pltpu.make_async_copy(
            tT_hbm.at[:, pl.ds(0, 512)], src_v.at[s], isems[s]).wait()

    def shuffle(s):
        def qbody(qq, _):
            for j in range(4):
                cols = jnp.full((_L,), 4 * qq + j, jnp.int32)
                for h in range(2):
                    dst_v[s, qq, pl.ds(32 * j + h * _L, _L)] = (
                        plsc.load_gather(src_v.at[s],
                                         [tok16 + h * _L, cols]))
            return 0
        lax.fori_loop(0, 128, qbody, 0)

    def wait_out(s):
        pltpu.make_async_copy(
            dst_v.at[s], t4_hbm.at[pl.ds(0, 128)], osems[s]).wait()

    fire_in(wid, 0)

    def body(i, _):
        for sb in range(2):
            k = i * 2 + sb
            c = k * _NW + wid
            cn = c + _NW

            @pl.when(c < _NSC)
            def _():
                wait_in(sb)
                @pl.when(cn < _NSC)
                def _():
                    fire_in(cn, 1 - sb)
                @pl.when(k >= 2)
                def _():
                    wait_out(sb)
                shuffle(sb)
                pltpu.async_copy(
                    dst_v.at[sb], t4_hbm.at[pl.ds(c * 128, 128)], osems[sb])
        return 0

    lax.fori_loop(0, (_SCPW + 1) // 2, body, 0)
    @pl.when(wid == 0)
    def _():
        pltpu.sync_copy(tail4_hbm, tail_v)
        wait_out(0)
        pltpu.sync_copy(tail_v, t4_hbm.at[pl.ds(_NSC * 128, 16)])
    @pl.when(wid != 0)
    def _():
        wait_out(0)
    wait_out(1)


def _emb_kernel(qT_hbm, cbT_hbm, table4_hbm, out_hbm,
                q_v, cb_v, g_v, o_v,
                gsem0, gsem1, gsem2, gsem3, osem0, osem1):
    wid = lax.axis_index("s") * _NC + lax.axis_index("c")
    col0 = wid * _BBLK
    gsems = (gsem0, gsem1, gsem2, gsem3)
    osems = (osem0, osem1)
    tok16 = lax.iota(jnp.int32, _L)

    pltpu.sync_copy(qT_hbm.at[:, pl.ds(col0, _BBLK)], q_v)
    pltpu.sync_copy(cbT_hbm.at[:, pl.ds(col0, _BBLK)], cb_v)

    def fire_gather(t, s):
        pltpu.async_copy(table4_hbm.at[q_v.at[t]], g_v.at[s], gsems[s])

    def wait_gather(s):
        pltpu.make_async_copy(
            table4_hbm.at[q_v.at[0]], g_v.at[s], gsems[s]).wait()

    def wait_out(s):
        pltpu.make_async_copy(
            o_v.at[s], out_hbm.at[0, :, pl.ds(col0, _BBLK)], osems[s]
        ).wait()

    for t in range(_NG - 1):
        fire_gather(t, t)

    def body(i, _):
        for b in range(_NG):
            t = i * _NG + b
            wait_gather(b)
            @pl.when(t + _NG - 1 < SEQ)
            def _():
                fire_gather(t + _NG - 1, (b + _NG - 1) % _NG)
            so = b % 2
            @pl.when(t >= 2)
            def _():
                wait_out(so)
            for tb in range(_BBLK // _L):
                cb = cb_v[t, pl.ds(tb * _L, _L)]
                rows = tok16 + (tb * _L)
                for d in range(EMBED_DIM):
                    o_v[so, d, pl.ds(tb * _L, _L)] = plsc.load_gather(
                        g_v.at[b], [rows, cb + d])
            pltpu.async_copy(
                o_v.at[so], out_hbm.at[t, :, pl.ds(col0, _BBLK)], osems[so]
            )
        return 0

    lax.fori_loop(0, SEQ // _NG, body, 0)
    wait_out(0)
    wait_out(1)


def kernel(x, table):
    assert x.shape == (BATCH, SEQ) and table.shape == (VOCAB, EMBED_DIM)
    xi = x.astype(jnp.int32)
    qT = (xi >> 2).T
    cbT = ((xi & 3) * EMBED_DIM).T
    tT = table.T

    mesh = plsc.VectorSubcoreMesh(core_axis_name="c", subcore_axis_name="s")
    cp = pltpu.CompilerParams(use_tc_tiling_on_sc=True,
                              needs_layout_passes=False)

    ka = functools.partial(
        pl.kernel, mesh=mesh,
        out_type=jax.ShapeDtypeStruct((VOCAB // 4, 128), jnp.float32),
        scratch_types=[
            pltpu.VMEM((2, EMBED_DIM, 512), jnp.float32),
            pltpu.VMEM((2, 128, 128), jnp.float32),
            pltpu.VMEM((16, 128), jnp.float32),
            pltpu.SemaphoreType.DMA,
            pltpu.SemaphoreType.DMA,
            pltpu.SemaphoreType.DMA,
            pltpu.SemaphoreType.DMA,
        ],
        compiler_params=cp,
    )(_repack_kernel)

    kb = functools.partial(
        pl.kernel, mesh=mesh,
        out_type=jax.ShapeDtypeStruct((SEQ, EMBED_DIM, BATCH), jnp.float32),
        scratch_types=[
            pltpu.VMEM((SEQ, _BBLK), jnp.int32),
            pltpu.VMEM((SEQ, _BBLK), jnp.int32),
            pltpu.VMEM((_NG, _BBLK, 128), jnp.float32),
            pltpu.VMEM((2, EMBED_DIM, _BBLK), jnp.float32),
            pltpu.SemaphoreType.DMA,
            pltpu.SemaphoreType.DMA,
            pltpu.SemaphoreType.DMA,
            pltpu.SemaphoreType.DMA,
            pltpu.SemaphoreType.DMA,
            pltpu.SemaphoreType.DMA,
        ],
        compiler_params=cp,
    )(_emb_kernel)

    tail4 = table[_NSC * 512:].reshape(16, 128)
    table4 = ka(tT, tail4)
    out = kb(qT, cbT, table4)
    return out.transpose(2, 0, 1)
